# Initial kernel scaffold; baseline (speedup 1.0000x reference)
import jax
import jax.numpy as jnp
from jax.experimental import pallas as pl


def kernel(x, w1, b1, w2, b2):
    raise NotImplementedError("write your pallas kernel here")


# speedup vs baseline: 1.2925x; 1.2925x over previous
import jax
import jax.numpy as jnp
from jax.experimental import pallas as pl
from jax.experimental.pallas import tpu as pltpu

_C = 256
_M = 32
_BB = 4
_VMEM = 56 * 1024 * 1024


def _se_block_kernel(x_ref, w1t_ref, b1_ref, w2t_ref, b2_ref, o_ref, *, inv_l):
    x = x_ref[...]
    pooled = jnp.sum(x, axis=2) * inv_l
    h = jnp.dot(pooled, w1t_ref[...],
                precision=jax.lax.Precision.HIGHEST,
                preferred_element_type=jnp.float32)
    h = jnp.maximum(h + b1_ref[...], 0.0)
    s = jnp.dot(h, w2t_ref[...],
                precision=jax.lax.Precision.HIGHEST,
                preferred_element_type=jnp.float32)
    s = jax.nn.sigmoid(s + b2_ref[...])
    o_ref[...] = x * s[:, :, None]


def kernel(x, w1, b1, w2, b2):
    B, C, L = x.shape
    bb = _BB if B % _BB == 0 else 1
    w1t = w1[:, :, 0].T.astype(jnp.float32)
    w2t = w2[:, :, 0].T.astype(jnp.float32)
    b1r = b1.astype(jnp.float32).reshape(1, _M)
    b2r = b2.astype(jnp.float32).reshape(1, _C)

    import functools
    return pl.pallas_call(
        functools.partial(_se_block_kernel, inv_l=1.0 / L),
        out_shape=jax.ShapeDtypeStruct((B, C, L), x.dtype),
        grid=(B // bb,),
        in_specs=[
            pl.BlockSpec((bb, C, L), lambda i: (i, 0, 0)),
            pl.BlockSpec((C, _M), lambda i: (0, 0)),
            pl.BlockSpec((1, _M), lambda i: (0, 0)),
            pl.BlockSpec((_M, C), lambda i: (0, 0)),
            pl.BlockSpec((1, _C), lambda i: (0, 0)),
        ],
        out_specs=pl.BlockSpec((bb, C, L), lambda i: (i, 0, 0)),
        compiler_params=pltpu.CompilerParams(
            dimension_semantics=("parallel",),
            vmem_limit_bytes=_VMEM),
    )(x, w1t, b1r, w2t, b2r)
